# baseline (speedup 1.0000x reference)
import functools

import jax
import jax.numpy as jnp
from jax import lax
from jax.experimental import pallas as pl
from jax.experimental.pallas import tpu as pltpu
from jax.experimental.pallas import tpu_sc as plsc

DIM = 768
NLANE = 16
NVEC = DIM // NLANE
NW = 32
CH = 40
EPS = 1e-5


def _lanesum(x):
    lanes = lax.iota(jnp.int32, NLANE)
    for k in (8, 4, 2, 1):
        x = x + x.at[lanes ^ k].get(mode="promise_in_bounds", unique_indices=True)
    return x


def _rsqrt16(x):
    i = plsc.bitcast(x, jnp.int32)
    y = plsc.bitcast(jnp.int32(0x5F3759DF) - (i >> 1), jnp.float32)
    for _ in range(3):
        y = y * (1.5 - 0.5 * x * y * y)
    return y


@functools.lru_cache(maxsize=None)
def _build(B, L):
    assert B % NW == 0 and L % CH == 0
    RPW = B // NW
    NCHUNK = L // CH
    mesh = plsc.VectorSubcoreMesh(core_axis_name="c", subcore_axis_name="s")

    @functools.partial(
        pl.kernel,
        mesh=mesh,
        compiler_params=pltpu.CompilerParams(needs_layout_passes=False),
        out_type=jax.ShapeDtypeStruct((B * L, DIM), jnp.float32),
        scratch_types=[
            pltpu.VMEM((CH,), jnp.int32),
            pltpu.VMEM((CH, DIM), jnp.float32),
            pltpu.VMEM((CH, DIM), jnp.float32),
            pltpu.VMEM((DIM,), jnp.float32),
            pltpu.VMEM((DIM,), jnp.float32),
            pltpu.SemaphoreType.DMA,
        ],
    )
    def body(ids_hbm, table_hbm, bias_hbm, gamma_hbm, beta_hbm, out_hbm,
             idx_v, rows_v, bias_v, gam_v, bet_v, sem):
        cid = lax.axis_index("c")
        sid = lax.axis_index("s")
        wid = sid * 2 + cid
        row0 = wid * RPW
        pltpu.sync_copy(gamma_hbm, gam_v)
        pltpu.sync_copy(beta_hbm, bet_v)

        def chunk_loop(lc, _):
            pltpu.sync_copy(bias_hbm.at[pl.ds(lc * CH, CH)], bias_v)

            def row_loop(r, _):
                base = (row0 + r) * L + lc * CH
                pltpu.sync_copy(ids_hbm.at[pl.ds(base, CH)], idx_v)
                pltpu.async_copy(table_hbm.at[idx_v], rows_v, sem).wait()

                def tok(t, _):
                    s = jnp.zeros((NLANE,), jnp.float32)
                    q = jnp.zeros((NLANE,), jnp.float32)
                    for j in range(NVEC):
                        sl = pl.ds(j * NLANE, NLANE)
                        x = rows_v[t, sl] + bias_v[t, sl]
                        rows_v[t, sl] = x
                        s = s + x
                        q = q + x * x
                    m16 = _lanesum(s) * (1.0 / DIM)
                    v16 = _lanesum(q) * (1.0 / DIM) - m16 * m16
                    r16 = _rsqrt16(v16 + EPS)
                    for j in range(NVEC):
                        sl = pl.ds(j * NLANE, NLANE)
                        x = rows_v[t, sl]
                        rows_v[t, sl] = (x - m16) * r16 * gam_v[sl] + bet_v[sl]
                    return 0

                lax.fori_loop(0, CH, tok, 0)
                pltpu.sync_copy(rows_v, out_hbm.at[pl.ds(base, CH)])
                return 0

            lax.fori_loop(0, RPW, row_loop, 0)
            return 0

        lax.fori_loop(0, NCHUNK, chunk_loop, 0)

    return body


def kernel(input_ids, table, pos_table, tt_table, gamma, beta):
    B, L = input_ids.shape
    bias = pos_table[:L] + tt_table[0][None, :]
    ids = input_ids.reshape(-1).astype(jnp.int32)
    out = _build(B, L)(ids, table, bias, gamma, beta)
    return out.reshape(B, L, DIM)
